# baseline (speedup 1.0000x reference)
import jax
import jax.numpy as jnp
from jax.experimental import pallas as pl
from jax.experimental.pallas import tpu as pltpu

B, L, D = 16, 4096, 1024
NSPLIT = 4
LSUB = L // NSPLIT
NTILE = D // 128
RC = 64


def _body(*refs):
    feats_refs = refs[:NSPLIT]
    out_ref = refs[NSPLIT]

    maxs = [None] * NTILE
    sums = [None] * NTILE
    first = True
    for s in range(NSPLIT):
        for r in range(LSUB // RC):
            rs = slice(r * RC, (r + 1) * RC)
            for j in range(NTILE):
                js = slice(j * 128, (j + 1) * 128)
                t = feats_refs[s][0, rs, js]
                bmax = jnp.max(t, axis=0)
                bsum = jnp.sum(t, axis=0)
                if first:
                    maxs[j] = bmax
                    sums[j] = bsum
                else:
                    maxs[j] = jnp.maximum(maxs[j], bmax)
                    sums[j] = sums[j] + bsum
            first = False
    inv = jnp.float32(1.0 / L)
    for j in range(NTILE):
        js = slice(j * 128, (j + 1) * 128)
        out_ref[0, 0, js] = maxs[j]
        out_ref[0, 0, D + j * 128:D + (j + 1) * 128] = sums[j] * inv


def kernel(feats, mask):
    in_specs = [
        pl.BlockSpec((1, LSUB, D), (lambda b, s=s: (b, s, 0)))
        for s in range(NSPLIT)
    ]
    out = pl.pallas_call(
        _body,
        grid=(B,),
        in_specs=in_specs,
        out_specs=pl.BlockSpec((1, 1, 2 * D), lambda b: (b, 0, 0)),
        out_shape=jax.ShapeDtypeStruct((B, 1, 2 * D), jnp.float32),
    )(*([feats] * NSPLIT))
    return out.reshape(B, 2 * D)


# speedup vs baseline: 1.0605x; 1.0605x over previous
import jax
import jax.numpy as jnp
from jax.experimental import pallas as pl
from jax.experimental.pallas import tpu as pltpu

B, L, D = 16, 4096, 1024
NSPLIT = 4
LSUB = L // NSPLIT
NTILE = D // 128
RC = 64


def _body(*refs):
    mask_refs = refs[:NSPLIT]
    feats_refs = refs[NSPLIT:2 * NSPLIT]
    out_ref = refs[2 * NSPLIT]

    cnt = jnp.float32(0.0)
    maxs = [None] * NTILE
    sums = [None] * NTILE
    first = True
    for s in range(NSPLIT):
        for r in range(LSUB // RC):
            rs = slice(r * RC, (r + 1) * RC)
            mf = mask_refs[s][0, rs, :].astype(jnp.float32)
            pen = (mf - jnp.float32(1.0)) * jnp.float32(1e30)
            cnt = cnt + jnp.sum(mf) * jnp.float32(1.0 / 128.0)
            for j in range(NTILE):
                js = slice(j * 128, (j + 1) * 128)
                t = feats_refs[s][0, rs, js] * mf
                bmax = jnp.max(t + pen, axis=0)
                bsum = jnp.sum(t, axis=0)
                if first:
                    maxs[j] = bmax
                    sums[j] = bsum
                else:
                    maxs[j] = jnp.maximum(maxs[j], bmax)
                    sums[j] = sums[j] + bsum
            first = False
    inv = jnp.float32(1.0) / cnt
    for j in range(NTILE):
        js = slice(j * 128, (j + 1) * 128)
        out_ref[0, 0, js] = maxs[j]
        out_ref[0, 0, D + j * 128:D + (j + 1) * 128] = sums[j] * inv


def kernel(feats, mask):
    mask128 = jnp.broadcast_to(
        mask[:, :, None], (B, L, 128)).astype(jnp.int8)
    in_specs = [
        pl.BlockSpec((1, LSUB, 128), (lambda b, s=s: (b, s, 0)))
        for s in range(NSPLIT)
    ] + [
        pl.BlockSpec((1, LSUB, D), (lambda b, s=s: (b, s, 0)))
        for s in range(NSPLIT)
    ]
    out = pl.pallas_call(
        _body,
        grid=(B,),
        in_specs=in_specs,
        out_specs=pl.BlockSpec((1, 1, 2 * D), lambda b: (b, 0, 0)),
        out_shape=jax.ShapeDtypeStruct((B, 1, 2 * D), jnp.float32),
        compiler_params=pltpu.CompilerParams(
            dimension_semantics=("parallel",)),
    )(*([mask128] * NSPLIT + [feats] * NSPLIT))
    return out.reshape(B, 2 * D)


# speedup vs baseline: 1.1145x; 1.0509x over previous
import jax
import jax.numpy as jnp
from jax import lax
from jax.experimental import pallas as pl
from jax.experimental.pallas import tpu as pltpu

B, L, D = 16, 4096, 1024
NBUF = 4
PARTS = 8
CH = L // PARTS
NCH = B * PARTS
NTILE = D // 128
RC = 64


def _body(mask_hbm, feats_hbm, out_ref, fbuf, mbuf, amax, asum, acnt,
          fsem, msem):
    def start(k, c):
        b = c // PARTS
        p = c % PARTS
        pltpu.make_async_copy(
            feats_hbm.at[b, pl.ds(p * CH, CH), :], fbuf.at[k], fsem.at[k]
        ).start()
        pltpu.make_async_copy(
            mask_hbm.at[b, pl.ds(p * CH, CH), :], mbuf.at[k], msem.at[k]
        ).start()

    for k in range(NBUF):
        start(k, k)

    amax[...] = jnp.full((1, D), -1e30, jnp.float32)
    asum[...] = jnp.zeros((1, D), jnp.float32)
    acnt[0] = jnp.float32(0.0)

    def group(g, _):
        for k in range(NBUF):
            c = g * NBUF + k
            pltpu.make_async_copy(
                feats_hbm.at[0, pl.ds(0, CH), :], fbuf.at[k], fsem.at[k]
            ).wait()
            pltpu.make_async_copy(
                mask_hbm.at[0, pl.ds(0, CH), :], mbuf.at[k], msem.at[k]
            ).wait()

            cnt = jnp.float32(0.0)
            maxs = [None] * NTILE
            sums = [None] * NTILE
            for r in range(CH // RC):
                rs = slice(r * RC, (r + 1) * RC)
                mf = mbuf[k, rs, :].astype(jnp.float32)
                pen = (mf - jnp.float32(1.0)) * jnp.float32(1e30)
                cnt = cnt + jnp.sum(mf) * jnp.float32(1.0 / 128.0)
                for j in range(NTILE):
                    js = slice(j * 128, (j + 1) * 128)
                    t = fbuf[k, rs, js] * mf
                    bmax = jnp.max(t + pen, axis=0)
                    bsum = jnp.sum(t, axis=0)
                    if r == 0:
                        maxs[j] = bmax
                        sums[j] = bsum
                    else:
                        maxs[j] = jnp.maximum(maxs[j], bmax)
                        sums[j] = sums[j] + bsum
            bmax_row = jnp.concatenate(maxs).reshape(1, D)
            bsum_row = jnp.concatenate(sums).reshape(1, D)
            amax[...] = jnp.maximum(amax[...], bmax_row)
            asum[...] = asum[...] + bsum_row
            acnt[0] = acnt[0] + cnt

            nxt = c + NBUF

            @pl.when(nxt < NCH)
            def _():
                start(k, nxt)

            @pl.when(c % PARTS == PARTS - 1)
            def _():
                b = c // PARTS
                inv = jnp.float32(1.0) / acnt[0]
                out_ref[pl.ds(b, 1), :D] = amax[...]
                out_ref[pl.ds(b, 1), D:] = asum[...] * inv
                amax[...] = jnp.full((1, D), -1e30, jnp.float32)
                asum[...] = jnp.zeros((1, D), jnp.float32)
                acnt[0] = jnp.float32(0.0)

        return ()

    lax.fori_loop(0, NCH // NBUF, group, (), unroll=False)


def kernel(feats, mask):
    mask128 = jnp.broadcast_to(
        mask[:, :, None], (B, L, 128)).astype(jnp.int8)
    out = pl.pallas_call(
        _body,
        in_specs=[
            pl.BlockSpec(memory_space=pltpu.MemorySpace.HBM),
            pl.BlockSpec(memory_space=pltpu.MemorySpace.HBM),
        ],
        out_specs=pl.BlockSpec(memory_space=pltpu.MemorySpace.VMEM),
        out_shape=jax.ShapeDtypeStruct((B, 2 * D), jnp.float32),
        scratch_shapes=[
            pltpu.VMEM((NBUF, CH, D), jnp.float32),
            pltpu.VMEM((NBUF, CH, 128), jnp.int8),
            pltpu.VMEM((1, D), jnp.float32),
            pltpu.VMEM((1, D), jnp.float32),
            pltpu.SMEM((1,), jnp.float32),
            pltpu.SemaphoreType.DMA((NBUF,)),
            pltpu.SemaphoreType.DMA((NBUF,)),
        ],
    )(mask128, feats)
    return out
